# baseline (speedup 1.0000x reference)
import functools

import jax
import jax.numpy as jnp
from jax import lax
from jax.experimental import pallas as pl
from jax.experimental.pallas import tpu as pltpu
from jax.experimental.pallas import tpu_sc as plsc

B = 4096
L = 50
D = 32
N = B * L
NC = 2
NS = 16
NW = NC * NS
PER_W = N // NW
G = 128
NG = PER_W // G


def _make_gather(V):
    mesh = plsc.VectorSubcoreMesh(core_axis_name="c", subcore_axis_name="s")

    @functools.partial(
        pl.kernel,
        out_type=jax.ShapeDtypeStruct((N, D), jnp.float32),
        mesh=mesh,
        scratch_types=[
            pltpu.VMEM((NG, G), jnp.int32),
            pltpu.VMEM((G, D), jnp.float32),
            pltpu.SemaphoreType.DMA,
        ],
        compiler_params=pltpu.CompilerParams(use_tc_tiling_on_sc=False),
    )
    def gather_kernel(idx_hbm, table_hbm, out_hbm, idx_v, rows_v, sem):
        wid = lax.axis_index("s") * NC + lax.axis_index("c")
        base = wid * PER_W
        pltpu.sync_copy(idx_hbm.at[wid], idx_v)

        def body(j, carry):
            pltpu.async_copy(table_hbm.at[idx_v.at[j]], rows_v, sem).wait()
            pltpu.sync_copy(rows_v, out_hbm.at[pl.ds(base + j * G, G)])
            return carry

        lax.fori_loop(0, NG, body, 0, unroll=False)

    return gather_kernel


def kernel(indices, table):
    idx = indices.astype(jnp.int32).reshape(NW, NG, G)
    out = _make_gather(table.shape[0])(idx, table)
    word_embeddings = out.reshape(B, L, D)
    lengths = jnp.full((B,), L, dtype=jnp.int32)
    return (word_embeddings, lengths)


# speedup vs baseline: 1.0401x; 1.0401x over previous
import functools

import jax
import jax.numpy as jnp
from jax import lax
from jax.experimental import pallas as pl
from jax.experimental.pallas import tpu as pltpu
from jax.experimental.pallas import tpu_sc as plsc

B = 4096
L = 50
D = 32
N = B * L
NC = 2
NS = 16
NW = NC * NS
PER_W = N // NW
G = 128
NG = PER_W // G
K = 10
CH = K * G
NCH = NG // K


def _make_gather(V):
    mesh = plsc.VectorSubcoreMesh(core_axis_name="c", subcore_axis_name="s")

    @functools.partial(
        pl.kernel,
        out_type=jax.ShapeDtypeStruct((N, D), jnp.float32),
        mesh=mesh,
        scratch_types=[
            pltpu.VMEM((NG, G), jnp.int32),
            pltpu.VMEM((2, CH, D), jnp.float32),
            pltpu.SemaphoreType.DMA,
            pltpu.SemaphoreType.DMA,
        ],
        compiler_params=pltpu.CompilerParams(use_tc_tiling_on_sc=False),
    )
    def gather_kernel(idx_hbm, table_hbm, out_hbm, idx_v, rows_v, sem_g, sem_st):
        wid = lax.axis_index("s") * NC + lax.axis_index("c")
        base = wid * PER_W
        pltpu.sync_copy(idx_hbm.at[wid], idx_v)

        def chunk_body(c, carry):
            slot = lax.rem(c, 2)
            buf = rows_v.at[slot]

            @pl.when(c >= 2)
            def _():
                pltpu.make_async_copy(buf, out_hbm.at[pl.ds(base, CH)], sem_st).wait()

            copies = [
                pltpu.async_copy(
                    table_hbm.at[idx_v.at[c * K + b]],
                    buf.at[pl.ds(b * G, G)],
                    sem_g,
                )
                for b in range(K)
            ]
            for cp in copies:
                cp.wait()

            pltpu.async_copy(buf, out_hbm.at[pl.ds(base + c * CH, CH)], sem_st)
            return carry

        lax.fori_loop(0, NCH, chunk_body, 0, unroll=False)

        pltpu.make_async_copy(rows_v.at[0], out_hbm.at[pl.ds(base, CH)], sem_st).wait()
        pltpu.make_async_copy(rows_v.at[1], out_hbm.at[pl.ds(base, CH)], sem_st).wait()

    return gather_kernel


def kernel(indices, table):
    idx = indices.astype(jnp.int32).reshape(NW, NG, G)
    out = _make_gather(table.shape[0])(idx, table)
    word_embeddings = out.reshape(B, L, D)
    lengths = jnp.full((B,), L, dtype=jnp.int32)
    return (word_embeddings, lengths)


# speedup vs baseline: 1.2286x; 1.1812x over previous
import functools

import jax
import jax.numpy as jnp
from jax import lax
from jax.experimental import pallas as pl
from jax.experimental.pallas import tpu as pltpu
from jax.experimental.pallas import tpu_sc as plsc

B = 4096
L = 50
D = 32
NC = 2
NS = 16
NW = NC * NS
BPW = B // NW


def _make_gather(V):
    mesh = plsc.VectorSubcoreMesh(core_axis_name="c", subcore_axis_name="s")

    @functools.partial(
        pl.kernel,
        out_type=jax.ShapeDtypeStruct((B, L, D), jnp.float32),
        mesh=mesh,
        scratch_types=[
            pltpu.VMEM((L, BPW), jnp.int32),
            pltpu.VMEM((2, BPW, D), jnp.float32),
            pltpu.SemaphoreType.DMA,
            pltpu.SemaphoreType.DMA,
        ],
        compiler_params=pltpu.CompilerParams(use_tc_tiling_on_sc=False),
    )
    def gather_kernel(idx_hbm, table_hbm, out_hbm, idx_v, rows_v, sem_g, sem_st):
        wid = lax.axis_index("s") * NC + lax.axis_index("c")
        b0 = wid * BPW
        pltpu.sync_copy(idx_hbm.at[wid], idx_v)

        pltpu.async_copy(table_hbm.at[idx_v.at[0]], rows_v.at[0], sem_g)

        def body(l, carry):
            slot = lax.rem(l, 2)
            pltpu.make_async_copy(
                table_hbm.at[idx_v.at[l]], rows_v.at[slot], sem_g
            ).wait()

            @pl.when(l >= 1)
            def _():
                pltpu.make_async_copy(
                    rows_v.at[0], out_hbm.at[pl.ds(b0, BPW), 0], sem_st
                ).wait()

            @pl.when(l + 1 < L)
            def _():
                nslot = lax.rem(l + 1, 2)
                pltpu.async_copy(
                    table_hbm.at[idx_v.at[l + 1]], rows_v.at[nslot], sem_g
                )

            pltpu.async_copy(
                rows_v.at[slot], out_hbm.at[pl.ds(b0, BPW), l], sem_st
            )
            return carry

        lax.fori_loop(0, L, body, 0, unroll=False)

        pltpu.make_async_copy(
            rows_v.at[0], out_hbm.at[pl.ds(b0, BPW), 0], sem_st
        ).wait()

    return gather_kernel


def kernel(indices, table):
    idx3 = indices.astype(jnp.int32).reshape(NW, BPW, L).transpose(0, 2, 1)
    tflat = lax.optimization_barrier(table.reshape(-1))
    t2 = tflat.reshape(table.shape[0], D)
    out = _make_gather(table.shape[0])(idx3, t2)
    lengths = jnp.full((B,), L, dtype=jnp.int32)
    return (out, lengths)


# speedup vs baseline: 1.3416x; 1.0920x over previous
import functools

import jax
import jax.numpy as jnp
from jax import lax
from jax.experimental import pallas as pl
from jax.experimental.pallas import tpu as pltpu
from jax.experimental.pallas import tpu_sc as plsc

B = 4096
L = 50
D = 32
NC = 2
NS = 16
NW = NC * NS
BPW = B // NW
RB = 2048


def _make_tc_transpose(V):
    grid = -(-V // RB)

    def tbody(x_ref, o_ref):
        x = x_ref[...]
        o_ref[...] = jnp.concatenate(
            [x[:, 512 * j : 512 * (j + 1)].T for j in range(RB // 512)], axis=1
        )

    return pl.pallas_call(
        tbody,
        grid=(grid,),
        in_specs=[pl.BlockSpec((D, RB), lambda i: (0, i))],
        out_specs=pl.BlockSpec((512, 128), lambda i: (i, 0)),
        out_shape=jax.ShapeDtypeStruct((grid * 512, 128), jnp.float32),
    )


def _make_sc_gather(V):
    mesh = plsc.VectorSubcoreMesh(core_axis_name="c", subcore_axis_name="s")

    @functools.partial(
        pl.kernel,
        out_type=jax.ShapeDtypeStruct((L, D, B), jnp.float32),
        mesh=mesh,
        scratch_types=[
            pltpu.VMEM((L, BPW), jnp.int32),
            pltpu.VMEM((BPW, D), jnp.float32),
            pltpu.VMEM((BPW, D), jnp.float32),
            pltpu.VMEM((D, BPW), jnp.float32),
            pltpu.VMEM((D, BPW), jnp.float32),
            pltpu.SemaphoreType.DMA,
            pltpu.SemaphoreType.DMA,
        ],
        compiler_params=pltpu.CompilerParams(
            use_tc_tiling_on_sc=False, needs_layout_passes=False
        ),
    )
    def gather_kernel(
        idx_hbm, table_hbm, out_hbm, idx_v, rows0, rows1, t0, t1, sem_g, sem_st
    ):
        wid = lax.axis_index("s") * NC + lax.axis_index("c")
        b0 = wid * BPW
        pltpu.sync_copy(idx_hbm.at[wid], idx_v)

        pltpu.async_copy(table_hbm.at[idx_v.at[0]], rows0, sem_g)
        lanes = lax.iota(jnp.int32, 16)

        def step(l, rows, tbuf, rows_next):
            pltpu.make_async_copy(table_hbm.at[idx_v.at[l]], rows, sem_g).wait()

            @pl.when(l + 1 < L)
            def _():
                pltpu.async_copy(table_hbm.at[idx_v.at[l + 1]], rows_next, sem_g)

            @pl.when(l >= 2)
            def _():
                pltpu.make_async_copy(
                    t0, out_hbm.at[0, :, pl.ds(b0, BPW)], sem_st
                ).wait()

            def dbody(d, dcarry):
                dvec = jnp.zeros((16,), jnp.int32) + d
                for jb in range(BPW // 16):
                    vec = plsc.load_gather(rows, [lanes + jb * 16, dvec])
                    tbuf[d, pl.ds(jb * 16, 16)] = vec
                return dcarry

            lax.fori_loop(0, D, dbody, 0, unroll=False)

            pltpu.async_copy(tbuf, out_hbm.at[l, :, pl.ds(b0, BPW)], sem_st)

        def body(i, carry):
            step(2 * i, rows0, t0, rows1)
            step(2 * i + 1, rows1, t1, rows0)
            return carry

        lax.fori_loop(0, L // 2, body, 0, unroll=False)

        for _ in range(2):
            pltpu.make_async_copy(
                t0, out_hbm.at[0, :, pl.ds(b0, BPW)], sem_st
            ).wait()

    return gather_kernel


def kernel(indices, table):
    V = table.shape[0]
    idx = indices.astype(jnp.int32)
    pos = (idx // RB * 512 + idx % 512) * 4 + idx % RB // 512
    idx3 = pos.reshape(NW, BPW, L).transpose(0, 2, 1)
    tlin = _make_tc_transpose(V)(table.T)
    vp = tlin.shape[0] * (128 // D)
    t2 = tlin.reshape(-1).reshape(vp, D)
    out_t = _make_sc_gather(vp)(idx3, t2)
    out = out_t.transpose(2, 0, 1)
    lengths = jnp.full((B,), L, dtype=jnp.int32)
    return (out, lengths)


# speedup vs baseline: 1.7602x; 1.3120x over previous
import functools

import jax
import jax.numpy as jnp
from jax import lax
from jax.experimental import pallas as pl
from jax.experimental.pallas import tpu as pltpu
from jax.experimental.pallas import tpu_sc as plsc

B = 4096
L = 50
D = 32
NC = 2
NS = 16
NW = NC * NS
BPW = B // NW
RB = 2048
NR = B * L // BPW


def _make_tc_table_format(V):
    grid = -(-V // RB)
    cdims = (((0,), (0,)), ((), ()))

    def tbody(x_ref, o_ref):
        x = x_ref[...]
        x4 = jnp.concatenate(
            [x[:, 512 * j : 512 * (j + 1)] for j in range(RB // 512)], axis=0
        )
        o_ref[...] = lax.dot_general(
            x4, jnp.eye(128, dtype=jnp.float32), cdims,
            preferred_element_type=jnp.float32,
        )

    return pl.pallas_call(
        tbody,
        grid=(grid,),
        in_specs=[pl.BlockSpec((D, RB), lambda i: (0, i))],
        out_specs=pl.BlockSpec((512, 128), lambda i: (i, 0)),
        out_shape=jax.ShapeDtypeStruct((grid * 512, 128), jnp.float32),
        compiler_params=pltpu.CompilerParams(fuse_transposed_lhs_in_matmul=True),
    )


def _make_sc_gather(V):
    mesh = plsc.VectorSubcoreMesh(core_axis_name="c", subcore_axis_name="s")

    @functools.partial(
        pl.kernel,
        out_type=jax.ShapeDtypeStruct((NR, BPW, 128), jnp.float32),
        mesh=mesh,
        scratch_types=[
            pltpu.VMEM((L, BPW), jnp.int32),
            pltpu.VMEM((2, BPW, D), jnp.float32),
            pltpu.SemaphoreType.DMA,
            pltpu.SemaphoreType.DMA,
        ],
        compiler_params=pltpu.CompilerParams(
            use_tc_tiling_on_sc=False, needs_layout_passes=False
        ),
    )
    def gather_kernel(idx_hbm, table_hbm, out_hbm, idx_v, rows_v, sem_g, sem_st):
        wid = lax.axis_index("s") * NC + lax.axis_index("c")
        pltpu.sync_copy(idx_hbm.at[wid], idx_v)

        pltpu.async_copy(table_hbm.at[idx_v.at[0]], rows_v.at[0], sem_g)

        def body(l, carry):
            slot = lax.rem(l, 2)
            pltpu.make_async_copy(
                table_hbm.at[idx_v.at[l]], rows_v.at[slot], sem_g
            ).wait()

            @pl.when(l >= 1)
            def _():
                pltpu.make_async_copy(
                    rows_v.at[0], out_hbm.at[0, :, pl.ds(0, D)], sem_st
                ).wait()

            @pl.when(l + 1 < L)
            def _():
                pltpu.async_copy(
                    table_hbm.at[idx_v.at[l + 1]], rows_v.at[lax.rem(l + 1, 2)], sem_g
                )

            pltpu.async_copy(
                rows_v.at[slot], out_hbm.at[l * NW + wid, :, pl.ds(0, D)], sem_st
            )
            return carry

        lax.fori_loop(0, L, body, 0, unroll=False)

        pltpu.make_async_copy(
            rows_v.at[0], out_hbm.at[0, :, pl.ds(0, D)], sem_st
        ).wait()

    return gather_kernel


def _make_tc_out_format():
    cdims = (((0,), (0,)), ((), ()))

    def t2body(x_ref, o_ref):
        eye = jnp.eye(BPW, dtype=jnp.float32)
        for w in range(NW):
            xw = x_ref[w, :, 0:D]
            o_ref[0, :, 128 * w : 128 * (w + 1)] = lax.dot_general(
                xw, eye, cdims, preferred_element_type=jnp.float32
            )

    return pl.pallas_call(
        t2body,
        grid=(L,),
        in_specs=[pl.BlockSpec((NW, BPW, 128), lambda i: (i, 0, 0))],
        out_specs=pl.BlockSpec((1, D, B), lambda i: (i, 0, 0)),
        out_shape=jax.ShapeDtypeStruct((L, D, B), jnp.float32),
        compiler_params=pltpu.CompilerParams(fuse_transposed_lhs_in_matmul=True),
    )


def kernel(indices, table):
    V = table.shape[0]
    idx = indices.astype(jnp.int32)
    pos = (idx // RB * 512 + idx % 512) * 4 + idx % RB // 512
    idx3 = pos.reshape(NW, BPW, L).transpose(0, 2, 1)
    tlin = _make_tc_table_format(V)(table.T)
    vp = tlin.shape[0] * (128 // D)
    t2 = tlin.reshape(-1).reshape(vp, D)
    staged = _make_sc_gather(vp)(idx3, t2)
    out_t = _make_tc_out_format()(staged)
    out = out_t.transpose(2, 0, 1)
    lengths = jnp.full((B,), L, dtype=jnp.int32)
    return (out, lengths)


# speedup vs baseline: 2.4727x; 1.4048x over previous
import functools

import jax
import jax.numpy as jnp
from jax import lax
from jax.experimental import pallas as pl
from jax.experimental.pallas import tpu as pltpu
from jax.experimental.pallas import tpu_sc as plsc

B = 4096
L = 50
D = 32
NC = 2
NS = 16
NW = NC * NS
BPW = B // NW
RB = 4096
NR = B * L // BPW


def _make_tc_table_format(V):
    grid = -(-V // RB)
    cdims = (((0,), (0,)), ((), ()))

    def tbody(x_ref, o_ref):
        x = x_ref[...]
        eye = jnp.eye(128, dtype=jnp.float32)
        for h in range(RB // 2048):
            x4 = jnp.concatenate(
                [
                    x[:, 2048 * h + 512 * j : 2048 * h + 512 * (j + 1)]
                    for j in range(4)
                ],
                axis=0,
            )
            o_ref[pl.ds(512 * h, 512), :] = lax.dot_general(
                x4, eye, cdims, preferred_element_type=jnp.float32,
            )

    return pl.pallas_call(
        tbody,
        grid=(grid,),
        in_specs=[pl.BlockSpec((D, RB), lambda i: (0, i))],
        out_specs=pl.BlockSpec((RB // 4, 128), lambda i: (i, 0)),
        out_shape=jax.ShapeDtypeStruct((grid * RB // 4, 128), jnp.float32),
        compiler_params=pltpu.CompilerParams(fuse_transposed_lhs_in_matmul=True),
    )


def _make_sc_gather(V):
    mesh = plsc.VectorSubcoreMesh(core_axis_name="c", subcore_axis_name="s")

    @functools.partial(
        pl.kernel,
        out_type=jax.ShapeDtypeStruct((NR, BPW, 128), jnp.float32),
        mesh=mesh,
        scratch_types=[
            pltpu.VMEM((L, BPW), jnp.int32),
            pltpu.VMEM((2, BPW, D), jnp.float32),
            pltpu.SemaphoreType.DMA,
            pltpu.SemaphoreType.DMA,
        ],
        compiler_params=pltpu.CompilerParams(
            use_tc_tiling_on_sc=False, needs_layout_passes=False
        ),
    )
    def gather_kernel(idx_hbm, table_hbm, out_hbm, idx_v, rows_v, sem_g, sem_st):
        wid = lax.axis_index("s") * NC + lax.axis_index("c")
        pltpu.sync_copy(idx_hbm.at[wid], idx_v)

        pltpu.async_copy(table_hbm.at[idx_v.at[0]], rows_v.at[0], sem_g)

        def body(l, carry):
            slot = lax.rem(l, 2)
            pltpu.make_async_copy(
                table_hbm.at[idx_v.at[l]], rows_v.at[slot], sem_g
            ).wait()

            @pl.when(l >= 1)
            def _():
                pltpu.make_async_copy(
                    rows_v.at[0], out_hbm.at[0, :, pl.ds(0, D)], sem_st
                ).wait()

            @pl.when(l + 1 < L)
            def _():
                pltpu.async_copy(
                    table_hbm.at[idx_v.at[l + 1]], rows_v.at[lax.rem(l + 1, 2)], sem_g
                )

            pltpu.async_copy(
                rows_v.at[slot], out_hbm.at[l * NW + wid, :, pl.ds(0, D)], sem_st
            )
            return carry

        lax.fori_loop(0, L, body, 0, unroll=False)

        pltpu.make_async_copy(
            rows_v.at[0], out_hbm.at[0, :, pl.ds(0, D)], sem_st
        ).wait()

    return gather_kernel


def _make_tc_out_format():
    cdims = (((0,), (0,)), ((), ()))

    def t2body(x_ref, o_ref):
        eye = jnp.eye(BPW, dtype=jnp.float32)
        for w in range(NW):
            xw = x_ref[w, :, 0:D]
            o_ref[0, :, 128 * w : 128 * (w + 1)] = lax.dot_general(
                xw, eye, cdims, preferred_element_type=jnp.float32,
            )

    return pl.pallas_call(
        t2body,
        grid=(L,),
        in_specs=[pl.BlockSpec((NW, BPW, 128), lambda i: (i, 0, 0))],
        out_specs=pl.BlockSpec((1, D, B), lambda i: (i, 0, 0)),
        out_shape=jax.ShapeDtypeStruct((L, D, B), jnp.float32),
        compiler_params=pltpu.CompilerParams(fuse_transposed_lhs_in_matmul=True),
    )


def kernel(indices, table):
    V = table.shape[0]
    idx = indices.astype(jnp.int32)
    pos = (idx // 2048 * 512 + idx % 512) * 4 + idx % 2048 // 512
    idx3 = pos.reshape(NW, BPW, L).transpose(0, 2, 1)
    tlin = _make_tc_table_format(V)(table.T)
    vp = tlin.shape[0] * (128 // D)
    t2 = tlin.reshape(-1).reshape(vp, D)
    staged = _make_sc_gather(vp)(idx3, t2)
    out_t = _make_tc_out_format()(staged)
    out = out_t.transpose(2, 0, 1)
    lengths = jnp.full((B,), L, dtype=jnp.int32)
    return (out, lengths)


# speedup vs baseline: 2.5189x; 1.0187x over previous
import functools

import jax
import jax.numpy as jnp
from jax import lax
from jax.experimental import pallas as pl
from jax.experimental.pallas import tpu as pltpu
from jax.experimental.pallas import tpu_sc as plsc

B = 4096
L = 50
D = 32
NC = 2
NS = 16
NW = NC * NS
BPW = B // NW
RB = 4096
NR = B * L // BPW


def _make_tc_table_format(V):
    grid = -(-V // RB)
    cdims = (((0,), (0,)), ((), ()))

    def tbody(x_ref, o_ref):
        x = x_ref[...]
        eye = jnp.eye(128, dtype=jnp.float32)
        for h in range(RB // 2048):
            x4 = jnp.concatenate(
                [
                    x[:, 2048 * h + 512 * j : 2048 * h + 512 * (j + 1)]
                    for j in range(4)
                ],
                axis=0,
            )
            o_ref[pl.ds(512 * h, 512), :] = lax.dot_general(
                x4, eye, cdims, preferred_element_type=jnp.float32,
            )

    return pl.pallas_call(
        tbody,
        grid=(grid,),
        in_specs=[pl.BlockSpec((D, RB), lambda i: (0, i))],
        out_specs=pl.BlockSpec((RB // 4, 128), lambda i: (i, 0)),
        out_shape=jax.ShapeDtypeStruct((grid * RB // 4, 128), jnp.float32),
        compiler_params=pltpu.CompilerParams(fuse_transposed_lhs_in_matmul=True),
    )


def _make_sc_gather(V):
    mesh = plsc.VectorSubcoreMesh(core_axis_name="c", subcore_axis_name="s")

    @functools.partial(
        pl.kernel,
        out_type=jax.ShapeDtypeStruct((NR // 4, BPW, 128), jnp.float32),
        mesh=mesh,
        scratch_types=[
            pltpu.VMEM((L, BPW), jnp.int32),
            pltpu.VMEM((2, BPW, D), jnp.float32),
            pltpu.SemaphoreType.DMA,
            pltpu.SemaphoreType.DMA,
        ],
        compiler_params=pltpu.CompilerParams(
            use_tc_tiling_on_sc=False, needs_layout_passes=False
        ),
    )
    def gather_kernel(idx_hbm, table_hbm, out_hbm, idx_v, rows_v, sem_g, sem_st):
        wid = lax.axis_index("s") * NC + lax.axis_index("c")
        pltpu.sync_copy(idx_hbm.at[wid], idx_v)

        pltpu.async_copy(table_hbm.at[idx_v.at[0]], rows_v.at[0], sem_g)

        def body(l, carry):
            slot = lax.rem(l, 2)
            pltpu.make_async_copy(
                table_hbm.at[idx_v.at[l]], rows_v.at[slot], sem_g
            ).wait()

            @pl.when(l >= 1)
            def _():
                pltpu.make_async_copy(
                    rows_v.at[0], out_hbm.at[0, :, pl.ds(0, D)], sem_st
                ).wait()

            @pl.when(l + 1 < L)
            def _():
                pltpu.async_copy(
                    table_hbm.at[idx_v.at[l + 1]], rows_v.at[lax.rem(l + 1, 2)], sem_g
                )

            c = l * NW + wid
            pltpu.async_copy(
                rows_v.at[slot],
                out_hbm.at[c // 4, :, pl.ds(lax.rem(c, 4) * D, D)],
                sem_st,
            )
            return carry

        lax.fori_loop(0, L, body, 0, unroll=False)

        pltpu.make_async_copy(
            rows_v.at[0], out_hbm.at[0, :, pl.ds(0, D)], sem_st
        ).wait()

    return gather_kernel


def _make_tc_out_format():
    cdims = (((0,), (0,)), ((), ()))

    def t2body(x_ref, o_ref):
        eye = jnp.eye(BPW, dtype=jnp.float32)
        for w in range(NW):
            xw = x_ref[w // 4, :, D * (w % 4) : D * (w % 4 + 1)]
            o_ref[0, :, 128 * w : 128 * (w + 1)] = lax.dot_general(
                xw, eye, cdims, preferred_element_type=jnp.float32,
            )

    return pl.pallas_call(
        t2body,
        grid=(L,),
        in_specs=[pl.BlockSpec((NW // 4, BPW, 128), lambda i: (i, 0, 0))],
        out_specs=pl.BlockSpec((1, D, B), lambda i: (i, 0, 0)),
        out_shape=jax.ShapeDtypeStruct((L, D, B), jnp.float32),
        compiler_params=pltpu.CompilerParams(fuse_transposed_lhs_in_matmul=True),
    )


def kernel(indices, table):
    V = table.shape[0]
    idx = indices.astype(jnp.int32)
    pos = (idx // 2048 * 512 + idx % 512) * 4 + idx % 2048 // 512
    idx3 = pos.reshape(NW, BPW, L).transpose(0, 2, 1)
    tlin = _make_tc_table_format(V)(table.T)
    vp = tlin.shape[0] * (128 // D)
    t2 = tlin.reshape(-1).reshape(vp, D)
    staged = _make_sc_gather(vp)(idx3, t2)
    out_t = _make_tc_out_format()(staged)
    out = out_t.transpose(2, 0, 1)
    lengths = jnp.full((B,), L, dtype=jnp.int32)
    return (out, lengths)


# speedup vs baseline: 2.7455x; 1.0900x over previous
import functools

import jax
import jax.numpy as jnp
from jax import lax
from jax.experimental import pallas as pl
from jax.experimental.pallas import tpu as pltpu
from jax.experimental.pallas import tpu_sc as plsc

B = 4096
L = 50
D = 32
NC = 2
NS = 16
NW = NC * NS
BPW = B // NW
RB = 4096
NR = B * L // BPW


def _make_tc_table_format(V):
    grid = -(-V // RB)
    cdims = (((0,), (0,)), ((), ()))

    def tbody(x_ref, o_ref):
        x = x_ref[...]
        eye = jnp.eye(128, dtype=jnp.float32)
        for h in range(RB // 2048):
            x4 = jnp.concatenate(
                [
                    x[:, 2048 * h + 512 * j : 2048 * h + 512 * (j + 1)]
                    for j in range(4)
                ],
                axis=0,
            )
            o_ref[pl.ds(512 * h, 512), :] = lax.dot_general(
                x4, eye, cdims, preferred_element_type=jnp.float32,
            )

    return pl.pallas_call(
        tbody,
        grid=(grid,),
        in_specs=[pl.BlockSpec((D, RB), lambda i: (0, i))],
        out_specs=pl.BlockSpec((RB // 4, 128), lambda i: (i, 0)),
        out_shape=jax.ShapeDtypeStruct((grid * RB // 4, 128), jnp.float32),
        compiler_params=pltpu.CompilerParams(fuse_transposed_lhs_in_matmul=True),
    )


def _make_sc_gather(V):
    mesh = plsc.VectorSubcoreMesh(core_axis_name="c", subcore_axis_name="s")

    @functools.partial(
        pl.kernel,
        out_type=jax.ShapeDtypeStruct((NR // 4, BPW, 128), jnp.float32),
        mesh=mesh,
        scratch_types=[
            pltpu.VMEM((L, BPW), jnp.int32),
            pltpu.VMEM((4, BPW, D), jnp.float32),
            pltpu.SemaphoreType.DMA,
            pltpu.SemaphoreType.DMA,
        ],
        compiler_params=pltpu.CompilerParams(
            use_tc_tiling_on_sc=False, needs_layout_passes=False
        ),
    )
    def gather_kernel(idx_hbm, table_hbm, out_hbm, idx_v, rows_v, sem_g, sem_st):
        wid = lax.axis_index("s") * NC + lax.axis_index("c")
        pltpu.sync_copy(idx_hbm.at[wid], idx_v)

        for p in range(3):
            pltpu.async_copy(table_hbm.at[idx_v.at[p]], rows_v.at[p], sem_g)

        def body(l, carry):
            slot = lax.rem(l, 4)
            pltpu.make_async_copy(
                table_hbm.at[idx_v.at[l]], rows_v.at[slot], sem_g
            ).wait()

            @pl.when(l >= 1)
            def _():
                pltpu.make_async_copy(
                    rows_v.at[0], out_hbm.at[0, :, pl.ds(0, D)], sem_st
                ).wait()

            @pl.when(l + 3 < L)
            def _():
                pltpu.async_copy(
                    table_hbm.at[idx_v.at[l + 3]], rows_v.at[lax.rem(l + 3, 4)], sem_g
                )

            c = l * NW + wid
            pltpu.async_copy(
                rows_v.at[slot],
                out_hbm.at[c // 4, :, pl.ds(lax.rem(c, 4) * D, D)],
                sem_st,
            )
            return carry

        lax.fori_loop(0, L, body, 0, unroll=False)

        pltpu.make_async_copy(
            rows_v.at[0], out_hbm.at[0, :, pl.ds(0, D)], sem_st
        ).wait()

    return gather_kernel


def _make_tc_out_format():
    cdims = (((0,), (0,)), ((), ()))

    def t2body(x_ref, o_ref):
        eye = jnp.eye(BPW, dtype=jnp.float32)
        for w in range(NW):
            xw = x_ref[w // 4, :, D * (w % 4) : D * (w % 4 + 1)]
            o_ref[0, :, 128 * w : 128 * (w + 1)] = lax.dot_general(
                xw, eye, cdims, preferred_element_type=jnp.float32,
            )

    return pl.pallas_call(
        t2body,
        grid=(L,),
        in_specs=[pl.BlockSpec((NW // 4, BPW, 128), lambda i: (i, 0, 0))],
        out_specs=pl.BlockSpec((1, D, B), lambda i: (i, 0, 0)),
        out_shape=jax.ShapeDtypeStruct((L, D, B), jnp.float32),
        compiler_params=pltpu.CompilerParams(fuse_transposed_lhs_in_matmul=True),
    )


def kernel(indices, table):
    V = table.shape[0]
    idx = indices.astype(jnp.int32)
    pos = (idx // 2048 * 512 + idx % 512) * 4 + idx % 2048 // 512
    idx3 = pos.reshape(NW, BPW, L).transpose(0, 2, 1)
    tlin = _make_tc_table_format(V)(table.T)
    vp = tlin.shape[0] * (128 // D)
    t2 = tlin.reshape(-1).reshape(vp, D)
    staged = _make_sc_gather(vp)(idx3, t2)
    out_t = _make_tc_out_format()(staged)
    out = out_t.transpose(2, 0, 1)
    lengths = jnp.full((B,), L, dtype=jnp.int32)
    return (out, lengths)


# speedup vs baseline: 3.6772x; 1.3393x over previous
import functools

import jax
import jax.numpy as jnp
from jax import lax
from jax.experimental import pallas as pl
from jax.experimental.pallas import tpu as pltpu
from jax.experimental.pallas import tpu_sc as plsc

B = 4096
L = 50
D = 32
NC = 2
NS = 16
NW = NC * NS
BPW = B // NW
RB = 8192
NR = B * L // BPW


def _make_tc_table_format(V):
    grid = -(-V // RB)
    cdims = (((0,), (0,)), ((), ()))

    def tbody(x_ref, o_ref):
        x = x_ref[...]
        eye = jnp.eye(128, dtype=jnp.float32)
        for h in range(RB // 2048):
            x4 = jnp.concatenate(
                [
                    x[:, 2048 * h + 512 * j : 2048 * h + 512 * (j + 1)]
                    for j in range(4)
                ],
                axis=0,
            )
            o_ref[pl.ds(512 * h, 512), :] = lax.dot_general(
                x4, eye, cdims, preferred_element_type=jnp.float32,
            )

    return pl.pallas_call(
        tbody,
        grid=(grid,),
        in_specs=[pl.BlockSpec((D, RB), lambda i: (0, i))],
        out_specs=pl.BlockSpec((RB // 4, 128), lambda i: (i, 0)),
        out_shape=jax.ShapeDtypeStruct((grid * RB // 4, 128), jnp.float32),
        compiler_params=pltpu.CompilerParams(fuse_transposed_lhs_in_matmul=True),
    )


def _make_sc_gather(V):
    mesh = plsc.VectorSubcoreMesh(core_axis_name="c", subcore_axis_name="s")

    @functools.partial(
        pl.kernel,
        out_type=jax.ShapeDtypeStruct((NR // 4, BPW, 128), jnp.float32),
        mesh=mesh,
        scratch_types=[
            pltpu.VMEM((L, BPW), jnp.int32),
            pltpu.VMEM((4, BPW, D), jnp.float32),
            pltpu.SemaphoreType.DMA,
            pltpu.SemaphoreType.DMA,
        ],
        compiler_params=pltpu.CompilerParams(
            use_tc_tiling_on_sc=False, needs_layout_passes=False
        ),
    )
    def gather_kernel(idx_hbm, table_hbm, out_hbm, idx_v, rows_v, sem_g, sem_st):
        wid = lax.axis_index("s") * NC + lax.axis_index("c")
        pltpu.sync_copy(idx_hbm.at[wid], idx_v)

        for p in range(3):
            pltpu.async_copy(table_hbm.at[idx_v.at[p]], rows_v.at[p], sem_g)

        def body(l, carry):
            slot = lax.rem(l, 4)
            pltpu.make_async_copy(
                table_hbm.at[idx_v.at[l]], rows_v.at[slot], sem_g
            ).wait()

            @pl.when(l >= 1)
            def _():
                pltpu.make_async_copy(
                    rows_v.at[0], out_hbm.at[0, :, pl.ds(0, D)], sem_st
                ).wait()

            @pl.when(l + 3 < L)
            def _():
                pltpu.async_copy(
                    table_hbm.at[idx_v.at[l + 3]], rows_v.at[lax.rem(l + 3, 4)], sem_g
                )

            c = l * NW + wid
            pltpu.async_copy(
                rows_v.at[slot],
                out_hbm.at[c // 4, :, pl.ds(lax.rem(c, 4) * D, D)],
                sem_st,
            )
            return carry

        lax.fori_loop(0, L, body, 0, unroll=False)

        pltpu.make_async_copy(
            rows_v.at[0], out_hbm.at[0, :, pl.ds(0, D)], sem_st
        ).wait()

    return gather_kernel


def _make_tc_out_format():
    cdims = (((0,), (0,)), ((), ()))

    def t2body(x_ref, o_ref):
        eye = jnp.eye(BPW, dtype=jnp.float32)
        for g in range(NW // 4):
            xq = lax.dot_general(
                x_ref[g], eye, cdims, preferred_element_type=jnp.float32,
            )
            for m in range(4):
                w = g * 4 + m
                o_ref[0, :, 128 * w : 128 * (w + 1)] = xq[D * m : D * (m + 1), :]

    return pl.pallas_call(
        t2body,
        grid=(L,),
        in_specs=[pl.BlockSpec((NW // 4, BPW, 128), lambda i: (i, 0, 0))],
        out_specs=pl.BlockSpec((1, D, B), lambda i: (i, 0, 0)),
        out_shape=jax.ShapeDtypeStruct((L, D, B), jnp.float32),
        compiler_params=pltpu.CompilerParams(fuse_transposed_lhs_in_matmul=True),
    )


def kernel(indices, table):
    V = table.shape[0]
    idx = indices.astype(jnp.int32)
    pos = (idx // 2048 * 512 + idx % 512) * 4 + idx % 2048 // 512
    idx3 = pos.reshape(NW, BPW, L).transpose(0, 2, 1)
    tlin = _make_tc_table_format(V)(table.T)
    vp = tlin.shape[0] * (128 // D)
    t2 = tlin.reshape(-1).reshape(vp, D)
    staged = _make_sc_gather(vp)(idx3, t2)
    out_t = _make_tc_out_format()(staged)
    out = out_t.transpose(2, 0, 1)
    lengths = jnp.full((B,), L, dtype=jnp.int32)
    return (out, lengths)


# speedup vs baseline: 4.4979x; 1.2232x over previous
import functools

import jax
import jax.numpy as jnp
from jax import lax
from jax.experimental import pallas as pl
from jax.experimental.pallas import tpu as pltpu
from jax.experimental.pallas import tpu_sc as plsc

B = 4096
L = 50
D = 32
NC = 2
NS = 16
NW = NC * NS
BPW = B // NW
RB = 16384
NR = B * L // BPW


def _make_tc_table_format(V):
    grid = -(-V // RB)
    cdims = (((0,), (0,)), ((), ()))

    def tbody(x_ref, o_ref):
        x = x_ref[...]
        eye = jnp.eye(128, dtype=jnp.float32)
        for h in range(RB // 2048):
            x4 = jnp.concatenate(
                [
                    x[:, 2048 * h + 512 * j : 2048 * h + 512 * (j + 1)]
                    for j in range(4)
                ],
                axis=0,
            )
            o_ref[pl.ds(512 * h, 512), :] = lax.dot_general(
                x4, eye, cdims, preferred_element_type=jnp.float32,
            )

    return pl.pallas_call(
        tbody,
        grid=(grid,),
        in_specs=[pl.BlockSpec((D, RB), lambda i: (0, i))],
        out_specs=pl.BlockSpec((RB // 4, 128), lambda i: (i, 0)),
        out_shape=jax.ShapeDtypeStruct((grid * RB // 4, 128), jnp.float32),
        compiler_params=pltpu.CompilerParams(fuse_transposed_lhs_in_matmul=True),
    )


def _make_sc_gather(V):
    mesh = plsc.VectorSubcoreMesh(core_axis_name="c", subcore_axis_name="s")

    @functools.partial(
        pl.kernel,
        out_type=jax.ShapeDtypeStruct((NR // 4, BPW, 128), jnp.float32),
        mesh=mesh,
        scratch_types=[
            pltpu.VMEM((L, BPW), jnp.int32),
            pltpu.VMEM((4, BPW, D), jnp.float32),
            pltpu.SemaphoreType.DMA,
            pltpu.SemaphoreType.DMA,
        ],
        compiler_params=pltpu.CompilerParams(
            use_tc_tiling_on_sc=False, needs_layout_passes=False
        ),
    )
    def gather_kernel(idx_hbm, table_hbm, out_hbm, idx_v, rows_v, sem_g, sem_st):
        wid = lax.axis_index("s") * NC + lax.axis_index("c")
        pltpu.sync_copy(idx_hbm.at[wid], idx_v)

        for p in range(3):
            pltpu.async_copy(table_hbm.at[idx_v.at[p]], rows_v.at[p], sem_g)

        def body(l, carry):
            slot = lax.rem(l, 4)
            pltpu.make_async_copy(
                table_hbm.at[idx_v.at[l]], rows_v.at[slot], sem_g
            ).wait()

            @pl.when(l >= 1)
            def _():
                pltpu.make_async_copy(
                    rows_v.at[0], out_hbm.at[0, :, pl.ds(0, D)], sem_st
                ).wait()

            @pl.when(l + 3 < L)
            def _():
                pltpu.async_copy(
                    table_hbm.at[idx_v.at[l + 3]], rows_v.at[lax.rem(l + 3, 4)], sem_g
                )

            c = l * NW + wid
            pltpu.async_copy(
                rows_v.at[slot],
                out_hbm.at[c // 4, :, pl.ds(lax.rem(c, 4) * D, D)],
                sem_st,
            )
            return carry

        lax.fori_loop(0, L, body, 0, unroll=False)

        pltpu.make_async_copy(
            rows_v.at[0], out_hbm.at[0, :, pl.ds(0, D)], sem_st
        ).wait()

    return gather_kernel


def _make_tc_out_format():
    cdims = (((0,), (0,)), ((), ()))

    def t2body(x_ref, o_ref):
        eye = jnp.eye(BPW, dtype=jnp.float32)
        for g in range(NW // 4):
            xq = lax.dot_general(
                x_ref[g], eye, cdims, preferred_element_type=jnp.float32,
            )
            for m in range(4):
                w = g * 4 + m
                o_ref[0, :, 128 * w : 128 * (w + 1)] = xq[D * m : D * (m + 1), :]

    return pl.pallas_call(
        t2body,
        grid=(L,),
        in_specs=[pl.BlockSpec((NW // 4, BPW, 128), lambda i: (i, 0, 0))],
        out_specs=pl.BlockSpec((1, D, B), lambda i: (i, 0, 0)),
        out_shape=jax.ShapeDtypeStruct((L, D, B), jnp.float32),
        compiler_params=pltpu.CompilerParams(fuse_transposed_lhs_in_matmul=True),
    )


def kernel(indices, table):
    V = table.shape[0]
    idx = indices.astype(jnp.int32)
    pos = (idx // 2048 * 512 + idx % 512) * 4 + idx % 2048 // 512
    idx3 = pos.reshape(NW, BPW, L).transpose(0, 2, 1)
    tlin = _make_tc_table_format(V)(table.T)
    vp = tlin.shape[0] * (128 // D)
    t2 = tlin.reshape(-1).reshape(vp, D)
    staged = _make_sc_gather(vp)(idx3, t2)
    out_t = _make_tc_out_format()(staged)
    out = out_t.transpose(2, 0, 1)
    lengths = jnp.full((B,), L, dtype=jnp.int32)
    return (out, lengths)


# speedup vs baseline: 4.9041x; 1.0903x over previous
import functools

import jax
import jax.numpy as jnp
from jax import lax
from jax.experimental import pallas as pl
from jax.experimental.pallas import tpu as pltpu
from jax.experimental.pallas import tpu_sc as plsc

B = 4096
L = 50
D = 32
NC = 2
NS = 16
NW = NC * NS
BPW = B // NW
RB = 32768
NR = B * L // BPW


def _make_tc_table_format(V):
    grid = -(-V // RB)
    cdims = (((0,), (0,)), ((), ()))

    def tbody(x_ref, o_ref):
        x = x_ref[...]
        eye = jnp.eye(128, dtype=jnp.float32)
        for h in range(RB // 2048):
            x4 = jnp.concatenate(
                [
                    x[:, 2048 * h + 512 * j : 2048 * h + 512 * (j + 1)]
                    for j in range(4)
                ],
                axis=0,
            )
            o_ref[pl.ds(512 * h, 512), :] = lax.dot_general(
                x4, eye, cdims, preferred_element_type=jnp.float32,
            )

    return pl.pallas_call(
        tbody,
        grid=(grid,),
        in_specs=[pl.BlockSpec((D, RB), lambda i: (0, i))],
        out_specs=pl.BlockSpec((RB // 4, 128), lambda i: (i, 0)),
        out_shape=jax.ShapeDtypeStruct((grid * RB // 4, 128), jnp.float32),
        compiler_params=pltpu.CompilerParams(fuse_transposed_lhs_in_matmul=True),
    )


def _make_sc_gather(V):
    mesh = plsc.VectorSubcoreMesh(core_axis_name="c", subcore_axis_name="s")

    @functools.partial(
        pl.kernel,
        out_type=jax.ShapeDtypeStruct((NR // 4, BPW, 128), jnp.float32),
        mesh=mesh,
        scratch_types=[
            pltpu.VMEM((L, BPW), jnp.int32),
            pltpu.VMEM((4, BPW, D), jnp.float32),
            pltpu.SemaphoreType.DMA,
            pltpu.SemaphoreType.DMA,
        ],
        compiler_params=pltpu.CompilerParams(
            use_tc_tiling_on_sc=False, needs_layout_passes=False
        ),
    )
    def gather_kernel(idx_hbm, table_hbm, out_hbm, idx_v, rows_v, sem_g, sem_st):
        wid = lax.axis_index("s") * NC + lax.axis_index("c")
        pltpu.sync_copy(idx_hbm.at[wid], idx_v)

        for p in range(3):
            pltpu.async_copy(table_hbm.at[idx_v.at[p]], rows_v.at[p], sem_g)

        def body(l, carry):
            slot = lax.rem(l, 4)
            pltpu.make_async_copy(
                table_hbm.at[idx_v.at[l]], rows_v.at[slot], sem_g
            ).wait()

            @pl.when(l >= 1)
            def _():
                pltpu.make_async_copy(
                    rows_v.at[0], out_hbm.at[0, :, pl.ds(0, D)], sem_st
                ).wait()

            @pl.when(l + 3 < L)
            def _():
                pltpu.async_copy(
                    table_hbm.at[idx_v.at[l + 3]], rows_v.at[lax.rem(l + 3, 4)], sem_g
                )

            c = l * NW + wid
            pltpu.async_copy(
                rows_v.at[slot],
                out_hbm.at[c // 4, :, pl.ds(lax.rem(c, 4) * D, D)],
                sem_st,
            )
            return carry

        lax.fori_loop(0, L, body, 0, unroll=False)

        pltpu.make_async_copy(
            rows_v.at[0], out_hbm.at[0, :, pl.ds(0, D)], sem_st
        ).wait()

    return gather_kernel


def _make_tc_out_format():
    cdims = (((0,), (0,)), ((), ()))

    def t2body(x_ref, o_ref):
        eye = jnp.eye(BPW, dtype=jnp.float32)
        for g in range(NW // 4):
            xq = lax.dot_general(
                x_ref[g], eye, cdims, preferred_element_type=jnp.float32,
            )
            for m in range(4):
                w = g * 4 + m
                o_ref[0, :, 128 * w : 128 * (w + 1)] = xq[D * m : D * (m + 1), :]

    return pl.pallas_call(
        t2body,
        grid=(L,),
        in_specs=[pl.BlockSpec((NW // 4, BPW, 128), lambda i: (i, 0, 0))],
        out_specs=pl.BlockSpec((1, D, B), lambda i: (i, 0, 0)),
        out_shape=jax.ShapeDtypeStruct((L, D, B), jnp.float32),
        compiler_params=pltpu.CompilerParams(fuse_transposed_lhs_in_matmul=True),
    )


def kernel(indices, table):
    V = table.shape[0]
    idx = indices.astype(jnp.int32)
    pos = (idx // 2048 * 512 + idx % 512) * 4 + idx % 2048 // 512
    idx3 = pos.reshape(NW, BPW, L).transpose(0, 2, 1)
    tlin = _make_tc_table_format(V)(table.T)
    vp = tlin.shape[0] * (128 // D)
    t2 = tlin.reshape(-1).reshape(vp, D)
    staged = _make_sc_gather(vp)(idx3, t2)
    out_t = _make_tc_out_format()(staged)
    out = out_t.transpose(2, 0, 1)
    lengths = jnp.full((B,), L, dtype=jnp.int32)
    return (out, lengths)


# speedup vs baseline: 4.9453x; 1.0084x over previous
import functools

import jax
import jax.numpy as jnp
from jax import lax
from jax.experimental import pallas as pl
from jax.experimental.pallas import tpu as pltpu
from jax.experimental.pallas import tpu_sc as plsc

B = 4096
L = 50
D = 32
NC = 2
NS = 16
NW = NC * NS
BPW = B // NW
RB = 65536
NR = B * L // BPW


def _make_tc_table_format(V):
    grid = -(-V // RB)
    cdims = (((0,), (0,)), ((), ()))

    def tbody(x_ref, o_ref):
        x = x_ref[...]
        eye = jnp.eye(128, dtype=jnp.float32)
        for h in range(RB // 2048):
            x4 = jnp.concatenate(
                [
                    x[:, 2048 * h + 512 * j : 2048 * h + 512 * (j + 1)]
                    for j in range(4)
                ],
                axis=0,
            )
            o_ref[pl.ds(512 * h, 512), :] = lax.dot_general(
                x4, eye, cdims, preferred_element_type=jnp.float32,
            )

    return pl.pallas_call(
        tbody,
        grid=(grid,),
        in_specs=[pl.BlockSpec((D, RB), lambda i: (0, i))],
        out_specs=pl.BlockSpec((RB // 4, 128), lambda i: (i, 0)),
        out_shape=jax.ShapeDtypeStruct((grid * RB // 4, 128), jnp.float32),
        compiler_params=pltpu.CompilerParams(fuse_transposed_lhs_in_matmul=True),
    )


def _make_sc_gather(V):
    mesh = plsc.VectorSubcoreMesh(core_axis_name="c", subcore_axis_name="s")

    @functools.partial(
        pl.kernel,
        out_type=jax.ShapeDtypeStruct((NR // 4, BPW, 128), jnp.float32),
        mesh=mesh,
        scratch_types=[
            pltpu.VMEM((L, BPW), jnp.int32),
            pltpu.VMEM((4, BPW, D), jnp.float32),
            pltpu.SemaphoreType.DMA,
            pltpu.SemaphoreType.DMA,
        ],
        compiler_params=pltpu.CompilerParams(
            use_tc_tiling_on_sc=False, needs_layout_passes=False
        ),
    )
    def gather_kernel(idx_hbm, table_hbm, out_hbm, idx_v, rows_v, sem_g, sem_st):
        wid = lax.axis_index("s") * NC + lax.axis_index("c")
        pltpu.sync_copy(idx_hbm.at[wid], idx_v)

        for p in range(3):
            pltpu.async_copy(table_hbm.at[idx_v.at[p]], rows_v.at[p], sem_g)

        def body(l, carry):
            slot = lax.rem(l, 4)
            pltpu.make_async_copy(
                table_hbm.at[idx_v.at[l]], rows_v.at[slot], sem_g
            ).wait()

            @pl.when(l >= 1)
            def _():
                pltpu.make_async_copy(
                    rows_v.at[0], out_hbm.at[0, :, pl.ds(0, D)], sem_st
                ).wait()

            @pl.when(l + 3 < L)
            def _():
                pltpu.async_copy(
                    table_hbm.at[idx_v.at[l + 3]], rows_v.at[lax.rem(l + 3, 4)], sem_g
                )

            c = l * NW + wid
            pltpu.async_copy(
                rows_v.at[slot],
                out_hbm.at[c // 4, :, pl.ds(lax.rem(c, 4) * D, D)],
                sem_st,
            )
            return carry

        lax.fori_loop(0, L, body, 0, unroll=False)

        pltpu.make_async_copy(
            rows_v.at[0], out_hbm.at[0, :, pl.ds(0, D)], sem_st
        ).wait()

    return gather_kernel


def _make_tc_out_format():
    cdims = (((0,), (0,)), ((), ()))

    def t2body(x_ref, o_ref):
        eye = jnp.eye(BPW, dtype=jnp.float32)
        for g in range(NW // 4):
            xq = lax.dot_general(
                x_ref[g], eye, cdims, preferred_element_type=jnp.float32,
            )
            for m in range(4):
                w = g * 4 + m
                o_ref[0, :, 128 * w : 128 * (w + 1)] = xq[D * m : D * (m + 1), :]

    return pl.pallas_call(
        t2body,
        grid=(L,),
        in_specs=[pl.BlockSpec((NW // 4, BPW, 128), lambda i: (i, 0, 0))],
        out_specs=pl.BlockSpec((1, D, B), lambda i: (i, 0, 0)),
        out_shape=jax.ShapeDtypeStruct((L, D, B), jnp.float32),
        compiler_params=pltpu.CompilerParams(fuse_transposed_lhs_in_matmul=True),
    )


def kernel(indices, table):
    V = table.shape[0]
    idx = indices.astype(jnp.int32)
    pos = (idx // 2048 * 512 + idx % 512) * 4 + idx % 2048 // 512
    idx3 = pos.reshape(NW, BPW, L).transpose(0, 2, 1)
    tlin = _make_tc_table_format(V)(table.T)
    vp = tlin.shape[0] * (128 // D)
    t2 = tlin.reshape(-1).reshape(vp, D)
    staged = _make_sc_gather(vp)(idx3, t2)
    out_t = _make_tc_out_format()(staged)
    out = out_t.transpose(2, 0, 1)
    lengths = jnp.full((B,), L, dtype=jnp.int32)
    return (out, lengths)


# speedup vs baseline: 5.3318x; 1.0782x over previous
import functools

import jax
import jax.numpy as jnp
from jax import lax
from jax.experimental import pallas as pl
from jax.experimental.pallas import tpu as pltpu
from jax.experimental.pallas import tpu_sc as plsc

B = 4096
L = 50
D = 32
NC = 2
NS = 16
NW = NC * NS
BPW = B // NW
RB = 65536
NR = B * L // BPW


def _make_tc_table_format(V):
    grid = -(-V // RB)
    cdims = (((0,), (0,)), ((), ()))

    def tbody(x_ref, o_ref):
        x = x_ref[...]
        eye = jnp.eye(128, dtype=jnp.float32)
        for h in range(RB // 2048):
            x4 = jnp.concatenate(
                [
                    x[:, 2048 * h + 512 * j : 2048 * h + 512 * (j + 1)]
                    for j in range(4)
                ],
                axis=0,
            )
            o_ref[pl.ds(512 * h, 512), :] = lax.dot_general(
                x4, eye, cdims, preferred_element_type=jnp.float32,
            )

    return pl.pallas_call(
        tbody,
        grid=(grid,),
        in_specs=[pl.BlockSpec((D, RB), lambda i: (0, i))],
        out_specs=pl.BlockSpec((RB // 4, 128), lambda i: (i, 0)),
        out_shape=jax.ShapeDtypeStruct((grid * RB // 4, 128), jnp.float32),
        compiler_params=pltpu.CompilerParams(fuse_transposed_lhs_in_matmul=True),
    )


def _make_sc_gather(V):
    mesh = plsc.VectorSubcoreMesh(core_axis_name="c", subcore_axis_name="s")

    @functools.partial(
        pl.kernel,
        out_type=jax.ShapeDtypeStruct((NR // 4, BPW, 128), jnp.float32),
        mesh=mesh,
        scratch_types=[
            pltpu.VMEM((L, BPW), jnp.int32),
            pltpu.VMEM((4, BPW, D), jnp.float32),
            pltpu.SemaphoreType.DMA,
            pltpu.SemaphoreType.DMA,
        ],
        compiler_params=pltpu.CompilerParams(
            use_tc_tiling_on_sc=False, needs_layout_passes=False
        ),
    )
    def gather_kernel(idx_hbm, table_hbm, out_hbm, idx_v, rows_v, sem_g, sem_st):
        wid = lax.axis_index("s") * NC + lax.axis_index("c")
        pltpu.sync_copy(idx_hbm.at[wid], idx_v)

        for p in range(3):
            pltpu.async_copy(table_hbm.at[idx_v.at[p]], rows_v.at[p], sem_g)

        def body(l, carry):
            slot = lax.rem(l, 4)
            pltpu.make_async_copy(
                table_hbm.at[idx_v.at[l]], rows_v.at[slot], sem_g
            ).wait()

            @pl.when(l >= 1)
            def _():
                pltpu.make_async_copy(
                    rows_v.at[0], out_hbm.at[0, :, pl.ds(0, D)], sem_st
                ).wait()

            @pl.when(l + 3 < L)
            def _():
                pltpu.async_copy(
                    table_hbm.at[idx_v.at[l + 3]], rows_v.at[lax.rem(l + 3, 4)], sem_g
                )

            c = l * NW + wid
            pltpu.async_copy(
                rows_v.at[slot],
                out_hbm.at[c // 4, :, pl.ds(lax.rem(c, 4) * D, D)],
                sem_st,
            )
            return carry

        lax.fori_loop(0, L, body, 0, unroll=False)

        pltpu.make_async_copy(
            rows_v.at[0], out_hbm.at[0, :, pl.ds(0, D)], sem_st
        ).wait()

    return gather_kernel


def _make_tc_out_format():
    cdims = (((0,), (0,)), ((), ()))

    LPB = 2

    def t2body(x_ref, o_ref):
        eye = jnp.eye(BPW, dtype=jnp.float32)
        for g in range(LPB * NW // 4):
            xq = lax.dot_general(
                x_ref[g], eye, cdims, preferred_element_type=jnp.float32,
            )
            for m in range(4):
                w = (g % 8) * 4 + m
                o_ref[g // 8, :, 128 * w : 128 * (w + 1)] = (
                    xq[D * m : D * (m + 1), :]
                )

    return pl.pallas_call(
        t2body,
        grid=(L // LPB,),
        in_specs=[pl.BlockSpec((LPB * NW // 4, BPW, 128), lambda i: (i, 0, 0))],
        out_specs=pl.BlockSpec((LPB, D, B), lambda i: (i, 0, 0)),
        out_shape=jax.ShapeDtypeStruct((L, D, B), jnp.float32),
        compiler_params=pltpu.CompilerParams(fuse_transposed_lhs_in_matmul=True),
    )


def kernel(indices, table):
    V = table.shape[0]
    idx = indices.astype(jnp.int32)
    pos = (idx // 2048 * 512 + idx % 512) * 4 + idx % 2048 // 512
    idx3 = pos.reshape(NW, BPW, L).transpose(0, 2, 1)
    tlin = _make_tc_table_format(V)(table.T)
    vp = tlin.shape[0] * (128 // D)
    t2 = tlin.reshape(-1).reshape(vp, D)
    staged = _make_sc_gather(vp)(idx3, t2)
    out_t = _make_tc_out_format()(staged)
    out = out_t.transpose(2, 0, 1)
    lengths = jnp.full((B,), L, dtype=jnp.int32)
    return (out, lengths)


# speedup vs baseline: 5.7041x; 1.0698x over previous
import functools

import jax
import jax.numpy as jnp
from jax import lax
from jax.experimental import pallas as pl
from jax.experimental.pallas import tpu as pltpu
from jax.experimental.pallas import tpu_sc as plsc

B = 4096
L = 50
D = 32
NC = 2
NS = 16
NW = NC * NS
BPW = B // NW
RB = 65536
NR = B * L // BPW


def _make_tc_table_format(V):
    grid = -(-V // RB)
    cdims = (((0,), (0,)), ((), ()))

    def tbody(x_ref, o_ref):
        x = x_ref[...]
        eye = jnp.eye(128, dtype=jnp.float32)
        for h in range(RB // 2048):
            x4 = jnp.concatenate(
                [
                    x[:, 2048 * h + 512 * j : 2048 * h + 512 * (j + 1)]
                    for j in range(4)
                ],
                axis=0,
            )
            o_ref[pl.ds(512 * h, 512), :] = lax.dot_general(
                x4, eye, cdims, preferred_element_type=jnp.float32,
            )

    return pl.pallas_call(
        tbody,
        grid=(grid,),
        in_specs=[pl.BlockSpec((D, RB), lambda i: (0, i))],
        out_specs=pl.BlockSpec((RB // 4, 128), lambda i: (i, 0)),
        out_shape=jax.ShapeDtypeStruct((grid * RB // 4, 128), jnp.float32),
        compiler_params=pltpu.CompilerParams(fuse_transposed_lhs_in_matmul=True),
    )


def _make_sc_gather(V):
    mesh = plsc.VectorSubcoreMesh(core_axis_name="c", subcore_axis_name="s")

    @functools.partial(
        pl.kernel,
        out_type=jax.ShapeDtypeStruct((NR // 4, BPW, 128), jnp.float32),
        mesh=mesh,
        scratch_types=[
            pltpu.VMEM((L, BPW), jnp.int32),
            pltpu.VMEM((4, BPW, D), jnp.float32),
            pltpu.SemaphoreType.DMA,
            pltpu.SemaphoreType.DMA,
        ],
        compiler_params=pltpu.CompilerParams(
            use_tc_tiling_on_sc=False, needs_layout_passes=False
        ),
    )
    def gather_kernel(idx_hbm, table_hbm, out_hbm, idx_v, rows_v, sem_g, sem_st):
        wid = lax.axis_index("s") * NC + lax.axis_index("c")
        pltpu.sync_copy(idx_hbm.at[wid], idx_v)

        for p in range(3):
            pltpu.async_copy(table_hbm.at[idx_v.at[p]], rows_v.at[p], sem_g)

        def body(l, carry):
            slot = lax.rem(l, 4)
            pltpu.make_async_copy(
                table_hbm.at[idx_v.at[l]], rows_v.at[slot], sem_g
            ).wait()

            @pl.when(l >= 1)
            def _():
                pltpu.make_async_copy(
                    rows_v.at[0], out_hbm.at[0, :, pl.ds(0, D)], sem_st
                ).wait()

            @pl.when(l + 3 < L)
            def _():
                pltpu.async_copy(
                    table_hbm.at[idx_v.at[l + 3]], rows_v.at[lax.rem(l + 3, 4)], sem_g
                )

            c = l * NW + wid
            pltpu.async_copy(
                rows_v.at[slot],
                out_hbm.at[c // 4, :, pl.ds(lax.rem(c, 4) * D, D)],
                sem_st,
            )
            return carry

        lax.fori_loop(0, L, body, 0, unroll=False)

        pltpu.make_async_copy(
            rows_v.at[0], out_hbm.at[0, :, pl.ds(0, D)], sem_st
        ).wait()

    return gather_kernel


def _make_tc_out_format():
    cdims = (((0,), (0,)), ((), ()))

    LPB = 5

    def t2body(x_ref, o_ref):
        eye = jnp.eye(BPW, dtype=jnp.float32)
        for g in range(LPB * NW // 4):
            xq = lax.dot_general(
                x_ref[g], eye, cdims, preferred_element_type=jnp.float32,
            )
            for m in range(4):
                w = (g % 8) * 4 + m
                o_ref[g // 8, :, 128 * w : 128 * (w + 1)] = (
                    xq[D * m : D * (m + 1), :]
                )

    return pl.pallas_call(
        t2body,
        grid=(L // LPB,),
        in_specs=[pl.BlockSpec((LPB * NW // 4, BPW, 128), lambda i: (i, 0, 0))],
        out_specs=pl.BlockSpec((LPB, D, B), lambda i: (i, 0, 0)),
        out_shape=jax.ShapeDtypeStruct((L, D, B), jnp.float32),
        compiler_params=pltpu.CompilerParams(fuse_transposed_lhs_in_matmul=True),
    )


def kernel(indices, table):
    V = table.shape[0]
    idx = indices.astype(jnp.int32)
    pos = (idx // 2048 * 512 + idx % 512) * 4 + idx % 2048 // 512
    idx3 = pos.reshape(NW, BPW, L).transpose(0, 2, 1)
    tlin = _make_tc_table_format(V)(table.T)
    vp = tlin.shape[0] * (128 // D)
    t2 = tlin.reshape(-1).reshape(vp, D)
    staged = _make_sc_gather(vp)(idx3, t2)
    out_t = _make_tc_out_format()(staged)
    out = out_t.transpose(2, 0, 1)
    lengths = jnp.full((B,), L, dtype=jnp.int32)
    return (out, lengths)
